# baseline (speedup 1.0000x reference)
import functools

import jax
import jax.numpy as jnp
from jax import lax
from jax.experimental import pallas as pl
from jax.experimental.pallas import tpu as pltpu
from jax.experimental.pallas import tpu_sc as plsc


def _table_body(emb_ref, w_ref, b_ref, t_ref):
    acc = lax.dot_general(
        emb_ref[...],
        w_ref[...],
        dimension_numbers=(((1,), (1,)), ((), ())),
        preferred_element_type=jnp.float32,
    )
    t_ref[...] = jnp.tanh(acc + b_ref[...])


def _make_table(emb, w, b, opad):
    v = emb.shape[0]
    o = w.shape[0]
    w32 = jnp.pad(w, ((0, opad - o), (0, 0)))
    b32 = jnp.pad(b, (0, opad - o)).reshape(1, opad)
    return pl.pallas_call(
        _table_body,
        out_shape=jax.ShapeDtypeStruct((v, opad), jnp.float32),
    )(emb, w32, b32)


def _finish_body(i_ref, o_ref):
    blkb, seq, o = o_ref.shape
    opad = i_ref.shape[0] // (blkb * seq)
    v = i_ref[...].reshape(blkb * seq, opad)
    o_ref[...] = v[:, :o].reshape(blkb, seq, o)


def _finish(flat, bsz, seq, o, opad, blkb=16):
    n = bsz * seq
    return pl.pallas_call(
        _finish_body,
        grid=(bsz // blkb,),
        in_specs=[pl.BlockSpec((blkb * seq * opad,), lambda i: (i,))],
        out_specs=pl.BlockSpec((blkb, seq, o), lambda i: (i, 0, 0)),
        out_shape=jax.ShapeDtypeStruct((bsz, seq, o), jnp.float32),
    )(flat)


_NC, _NS = 2, 16
_NW = _NC * _NS


@functools.lru_cache(maxsize=None)
def _make_gather(bsz, seq, opad, cb):
    n = bsz * seq
    per_w = n // _NW
    bat_w = bsz // _NW
    chunk = cb * seq
    nchunk = per_w // chunk
    assert per_w % chunk == 0 and n % (8 * _NW) == 0

    mesh = plsc.VectorSubcoreMesh(core_axis_name="c", subcore_axis_name="s")

    @functools.partial(
        pl.kernel,
        mesh=mesh,
        compiler_params=pltpu.CompilerParams(use_tc_tiling_on_sc=False),
        out_type=jax.ShapeDtypeStruct((bsz, seq, opad), jnp.float32),
        scratch_types=[
            pltpu.VMEM((per_w,), jnp.int32),
            pltpu.VMEM((chunk, opad), jnp.float32),
            pltpu.VMEM((chunk, opad), jnp.float32),
            pltpu.VMEM_SHARED((500, opad), jnp.float32),
            pltpu.SemaphoreType.DMA,
            pltpu.SemaphoreType.DMA,
            pltpu.SemaphoreType.DMA,
            pltpu.SemaphoreType.DMA,
        ],
    )
    def _gather(table_hbm, idx_hbm, out_hbm, idx_v, rows0, rows1, table_sh,
                g0, g1, s0, s1):
        sid = lax.axis_index("s")
        wid = sid * _NC + lax.axis_index("c")
        base = wid * per_w
        bbase = wid * bat_w
        @pl.when(sid == 0)
        def _():
            pltpu.sync_copy(table_hbm, table_sh)

        pltpu.sync_copy(idx_hbm.at[pl.ds(base, per_w)], idx_v)
        plsc.subcore_barrier()

        bufs = (rows0, rows1)
        gsems = (g0, g1)
        ssems = (s0, s1)
        scat = [None] * nchunk
        for c in range(nchunk):
            bi = c & 1
            if c >= 2:
                for h in scat[c - 2]:
                    h.wait()
            gath = pltpu.async_copy(
                table_sh.at[idx_v.at[pl.ds(c * chunk, chunk)]], bufs[bi], gsems[bi]
            )
            gath.wait()
            scat[c] = [
                pltpu.async_copy(
                    bufs[bi].at[pl.ds(k * seq, seq)],
                    out_hbm.at[bbase + c * cb + k],
                    ssems[bi],
                )
                for k in range(cb)
            ]
        for c in (nchunk - 2, nchunk - 1):
            for h in scat[c]:
                h.wait()

    return _gather


def kernel(x, emb, W, b):
    bsz, seq = x.shape
    o = W.shape[0]
    opad = 32
    n = bsz * seq
    table = _make_table(emb, W, b, opad)
    idx = x.reshape(-1).astype(jnp.int32)
    out32 = _make_gather(bsz, seq, opad, 4)(table, idx)
    return out32[:, :, :o]


# speedup vs baseline: 1.0071x; 1.0071x over previous
import functools

import jax
import jax.numpy as jnp
from jax import lax
from jax.experimental import pallas as pl
from jax.experimental.pallas import tpu as pltpu
from jax.experimental.pallas import tpu_sc as plsc


def _table_body(emb_ref, w_ref, b_ref, t_ref):
    acc = lax.dot_general(
        emb_ref[...],
        w_ref[...],
        dimension_numbers=(((1,), (1,)), ((), ())),
        preferred_element_type=jnp.float32,
    )
    t_ref[...] = jnp.tanh(acc + b_ref[...])


def _make_table(emb, w, b, opad):
    v = emb.shape[0]
    o = w.shape[0]
    w32 = jnp.pad(w, ((0, opad - o), (0, 0)))
    b32 = jnp.pad(b, (0, opad - o)).reshape(1, opad)
    return pl.pallas_call(
        _table_body,
        out_shape=jax.ShapeDtypeStruct((v, opad), jnp.float32),
    )(emb, w32, b32)


def _finish_body(i_ref, o_ref):
    blkb, seq, o = o_ref.shape
    opad = i_ref.shape[0] // (blkb * seq)
    v = i_ref[...].reshape(blkb * seq, opad)
    o_ref[...] = v[:, :o].reshape(blkb, seq, o)


def _finish(flat, bsz, seq, o, opad, blkb=16):
    n = bsz * seq
    return pl.pallas_call(
        _finish_body,
        grid=(bsz // blkb,),
        in_specs=[pl.BlockSpec((blkb * seq * opad,), lambda i: (i,))],
        out_specs=pl.BlockSpec((blkb, seq, o), lambda i: (i, 0, 0)),
        out_shape=jax.ShapeDtypeStruct((bsz, seq, o), jnp.float32),
    )(flat)


_NC, _NS = 2, 16
_NW = _NC * _NS


@functools.lru_cache(maxsize=None)
def _make_gather(n, opad, chunk):
    per_w = n // _NW
    nchunk = per_w // chunk
    assert per_w % chunk == 0 and n % (8 * _NW) == 0

    mesh = plsc.VectorSubcoreMesh(core_axis_name="c", subcore_axis_name="s")

    @functools.partial(
        pl.kernel,
        mesh=mesh,
        compiler_params=pltpu.CompilerParams(use_tc_tiling_on_sc=False),
        out_type=jax.ShapeDtypeStruct((n, opad), jnp.float32),
        scratch_types=[
            pltpu.VMEM((per_w,), jnp.int32),
            pltpu.VMEM((chunk, opad), jnp.float32),
            pltpu.VMEM((chunk, opad), jnp.float32),
            pltpu.VMEM_SHARED((500, opad), jnp.float32),
            pltpu.SemaphoreType.DMA,
            pltpu.SemaphoreType.DMA,
            pltpu.SemaphoreType.DMA,
            pltpu.SemaphoreType.DMA,
        ],
    )
    def _gather(table_hbm, idx_hbm, out_hbm, idx_v, rows0, rows1, table_sh,
                g0, g1, s0, s1):
        sid = lax.axis_index("s")
        wid = sid * _NC + lax.axis_index("c")
        base = wid * per_w
        @pl.when(sid == 0)
        def _():
            pltpu.sync_copy(table_hbm, table_sh)

        pltpu.sync_copy(idx_hbm.at[pl.ds(base, per_w)], idx_v)
        plsc.subcore_barrier()

        bufs = (rows0, rows1)
        gsems = (g0, g1)
        ssems = (s0, s1)
        scat = [None] * nchunk
        for c in range(nchunk):
            bi = c & 1
            if c >= 2:
                scat[c - 2].wait()
            gath = pltpu.async_copy(
                table_sh.at[idx_v.at[pl.ds(c * chunk, chunk)]], bufs[bi], gsems[bi]
            )
            gath.wait()
            scat[c] = pltpu.async_copy(
                bufs[bi], out_hbm.at[pl.ds(base + c * chunk, chunk)], ssems[bi]
            )
        scat[nchunk - 2].wait()
        scat[nchunk - 1].wait()

    return _gather


def kernel(x, emb, W, b):
    bsz, seq = x.shape
    o = W.shape[0]
    opad = 32
    n = bsz * seq
    table = _make_table(emb, W, b, opad)
    idx = x.reshape(-1).astype(jnp.int32)
    out32 = _make_gather(n, opad, 1280)(table, idx)
    return out32[:, :o].reshape(bsz, seq, o)


# speedup vs baseline: 1.0119x; 1.0048x over previous
import functools

import jax
import jax.numpy as jnp
from jax import lax
from jax.experimental import pallas as pl
from jax.experimental.pallas import tpu as pltpu
from jax.experimental.pallas import tpu_sc as plsc


def _table_body(emb_ref, w_ref, b_ref, t_ref):
    o = w_ref.shape[0]
    acc = lax.dot_general(
        emb_ref[...],
        w_ref[...],
        dimension_numbers=(((1,), (1,)), ((), ())),
        preferred_element_type=jnp.float32,
    )
    t_ref[:, :o] = jnp.tanh(acc + b_ref[...])
    t_ref[:, o:] = jnp.zeros((t_ref.shape[0], t_ref.shape[1] - o), jnp.float32)


def _make_table(emb, w, b, opad):
    v = emb.shape[0]
    o = w.shape[0]
    return pl.pallas_call(
        _table_body,
        out_shape=jax.ShapeDtypeStruct((v, opad), jnp.float32),
    )(emb, w, b.reshape(1, o))


def _finish_body(i_ref, o_ref):
    blkb, seq, o = o_ref.shape
    opad = i_ref.shape[0] // (blkb * seq)
    v = i_ref[...].reshape(blkb * seq, opad)
    o_ref[...] = v[:, :o].reshape(blkb, seq, o)


def _finish(flat, bsz, seq, o, opad, blkb=16):
    n = bsz * seq
    return pl.pallas_call(
        _finish_body,
        grid=(bsz // blkb,),
        in_specs=[pl.BlockSpec((blkb * seq * opad,), lambda i: (i,))],
        out_specs=pl.BlockSpec((blkb, seq, o), lambda i: (i, 0, 0)),
        out_shape=jax.ShapeDtypeStruct((bsz, seq, o), jnp.float32),
    )(flat)


_NC, _NS = 2, 16
_NW = _NC * _NS


@functools.lru_cache(maxsize=None)
def _make_gather(n, opad, chunk):
    per_w = n // _NW
    nchunk = per_w // chunk
    assert per_w % chunk == 0 and n % (8 * _NW) == 0

    mesh = plsc.VectorSubcoreMesh(core_axis_name="c", subcore_axis_name="s")

    @functools.partial(
        pl.kernel,
        mesh=mesh,
        compiler_params=pltpu.CompilerParams(use_tc_tiling_on_sc=False),
        out_type=jax.ShapeDtypeStruct((n, opad), jnp.float32),
        scratch_types=[
            pltpu.VMEM((per_w,), jnp.int32),
            pltpu.VMEM((chunk, opad), jnp.float32),
            pltpu.VMEM((chunk, opad), jnp.float32),
            pltpu.VMEM_SHARED((500, opad), jnp.float32),
            pltpu.SemaphoreType.DMA,
            pltpu.SemaphoreType.DMA,
            pltpu.SemaphoreType.DMA,
            pltpu.SemaphoreType.DMA,
        ],
    )
    def _gather(table_hbm, idx_hbm, out_hbm, idx_v, rows0, rows1, table_sh,
                g0, g1, s0, s1):
        sid = lax.axis_index("s")
        wid = sid * _NC + lax.axis_index("c")
        base = wid * per_w
        @pl.when(sid == 0)
        def _():
            pltpu.sync_copy(table_hbm, table_sh)

        pltpu.sync_copy(idx_hbm.at[pl.ds(base, per_w)], idx_v)
        plsc.subcore_barrier()

        bufs = (rows0, rows1)
        gsems = (g0, g1)
        ssems = (s0, s1)
        scat = [None] * nchunk
        for c in range(nchunk):
            bi = c & 1
            if c >= 2:
                scat[c - 2].wait()
            gath = pltpu.async_copy(
                table_sh.at[idx_v.at[pl.ds(c * chunk, chunk)]], bufs[bi], gsems[bi]
            )
            gath.wait()
            scat[c] = pltpu.async_copy(
                bufs[bi], out_hbm.at[pl.ds(base + c * chunk, chunk)], ssems[bi]
            )
        scat[nchunk - 2].wait()
        scat[nchunk - 1].wait()

    return _gather


def kernel(x, emb, W, b):
    bsz, seq = x.shape
    o = W.shape[0]
    opad = 32
    n = bsz * seq
    table = _make_table(emb, W, b, opad)
    idx = x.reshape(-1).astype(jnp.int32)
    out32 = _make_gather(n, opad, 1600)(table, idx)
    return out32[:, :o].reshape(bsz, seq, o)
